# baseline (speedup 1.0000x reference)
import jax
import jax.numpy as jnp
from jax.experimental import pallas as pl
from jax.experimental.pallas import tpu as pltpu

_B, _N, _M = 8, 2048, 4096
_BETA, _GAMMA, _DELTA = 1.0, 1.0, 0.0


def _cd_kernel(x1_ref, x2t_ref, idx12_ref, cmin_ref, cidx_ref, loss_ref):
    b = pl.program_id(0)
    x1 = x1_ref[0]
    x2t = x2t_ref[0]
    a0, a1, a2 = x1[:, 0:1], x1[:, 1:2], x1[:, 2:3]
    aa = a0 * a0 + a1 * a1 + a2 * a2
    b0, b1, b2 = x2t[0:1, :], x2t[1:2, :], x2t[2:3, :]
    bb = b0 * b0 + b1 * b1 + b2 * b2
    ab = jax.lax.dot_general(
        x1, x2t, (((1,), (0,)), ((), ())), preferred_element_type=jnp.float32
    )
    d = aa + bb - 2.0 * ab

    C = 128
    lane128 = jax.lax.broadcasted_iota(jnp.int32, (_N, C), 1)
    rval = d[:, 0:C]
    ridx = lane128
    for c in range(1, _M // C):
        dc = d[:, c * C : (c + 1) * C]
        lt = dc < rval
        rval = jnp.where(lt, dc, rval)
        ridx = jnp.where(lt, lane128 + c * C, ridx)
    rmin = jnp.min(rval, axis=1, keepdims=True)
    ridx_f = jnp.min(jnp.where(rval == rmin, ridx, _M), axis=1, keepdims=True)
    idx12_ref[0, :, :] = ridx_f

    R = 8
    subl = jax.lax.broadcasted_iota(jnp.int32, (R, _M), 0)
    cval = d[0:R, :]
    cidx = subl
    for r in range(1, _N // R):
        dr = d[r * R : (r + 1) * R, :]
        lt = dr < cval
        cval = jnp.where(lt, dr, cval)
        cidx = jnp.where(lt, subl + r * R, cidx)
    cmin2 = jnp.min(cval, axis=0, keepdims=True)
    cidx_f = jnp.min(jnp.where(cval == cmin2, cidx, _N), axis=0)
    cmin_ref[0, 0, :] = cmin2[0]
    cidx_ref[0, 0, :] = cidx_f

    lane = jax.lax.broadcasted_iota(jnp.int32, (1, 128), 1)
    contrib = (
        jnp.sum(rmin) / _N
        + jnp.max(rmin) * _BETA
        + (_GAMMA + _DELTA * _N) * jnp.sum(cmin2) / _M
    ) * (1.0 / _B)
    contrib_vec = jnp.where(lane == 0, contrib, 0.0)

    @pl.when(b == 0)
    def _init():
        loss_ref[0, :, :] = contrib_vec

    @pl.when(b > 0)
    def _accum():
        loss_ref[0, :, :] = loss_ref[0, :, :] + contrib_vec


def kernel(xyz1, xyz2):
    xyz2t = xyz2.transpose(0, 2, 1)
    idx12_col, _cmin, cidx, loss_vec = pl.pallas_call(
        _cd_kernel,
        grid=(_B,),
        in_specs=[
            pl.BlockSpec((1, _N, 3), lambda b: (b, 0, 0)),
            pl.BlockSpec((1, 3, _M), lambda b: (b, 0, 0)),
        ],
        out_specs=[
            pl.BlockSpec((1, _N, 1), lambda b: (b, 0, 0)),
            pl.BlockSpec((1, 1, _M), lambda b: (b, 0, 0)),
            pl.BlockSpec((1, 1, _M), lambda b: (b, 0, 0)),
            pl.BlockSpec((1, 1, 128), lambda b: (0, 0, 0)),
        ],
        out_shape=[
            jax.ShapeDtypeStruct((_B, _N, 1), jnp.int32),
            jax.ShapeDtypeStruct((_B, 1, _M), jnp.float32),
            jax.ShapeDtypeStruct((_B, 1, _M), jnp.int32),
            jax.ShapeDtypeStruct((1, 1, 128), jnp.float32),
        ],
        compiler_params=pltpu.CompilerParams(
            dimension_semantics=("arbitrary",)
        ),
    )(xyz1, xyz2t)
    idx12 = idx12_col.reshape(_B, _N).astype(jnp.int64)
    idx21 = cidx.reshape(_B, _M).astype(jnp.int64)
    loss = loss_vec[0, 0, 0]
    return (loss, idx12, idx21)


# speedup vs baseline: 1.0090x; 1.0090x over previous
import jax
import jax.numpy as jnp
from jax.experimental import pallas as pl
from jax.experimental.pallas import tpu as pltpu

_B, _N, _M = 8, 2048, 4096
_BETA, _GAMMA, _DELTA = 1.0, 1.0, 0.0


def _cd_kernel(x1_ref, x2t_ref, idx12_ref, cmin_ref, cidx_ref, loss_ref):
    b = pl.program_id(0)
    x1 = x1_ref[0]
    x2t = x2t_ref[0]
    a0, a1, a2 = x1[:, 0:1], x1[:, 1:2], x1[:, 2:3]
    aa = a0 * a0 + a1 * a1 + a2 * a2
    b0, b1, b2 = x2t[0:1, :], x2t[1:2, :], x2t[2:3, :]
    bb = b0 * b0 + b1 * b1 + b2 * b2
    ab = jax.lax.dot_general(
        x1, x2t, (((1,), (0,)), ((), ())), preferred_element_type=jnp.float32
    )
    d = aa + bb - 2.0 * ab

    C = 128
    lane128 = jax.lax.broadcasted_iota(jnp.int32, (_N, C), 1)
    rval = d[:, 0:C]
    ridx = lane128
    for c in range(1, _M // C):
        dc = d[:, c * C : (c + 1) * C]
        lt = dc < rval
        rval = jnp.minimum(dc, rval)
        ridx = jnp.where(lt, lane128 + c * C, ridx)
    rmin = jnp.min(rval, axis=1, keepdims=True)
    ridx_f = jnp.min(jnp.where(rval == rmin, ridx, _M), axis=1, keepdims=True)
    idx12_ref[0, :, :] = ridx_f

    R = 8
    subl = jax.lax.broadcasted_iota(jnp.int32, (R, _M), 0)
    cval = d[0:R, :]
    cidx = subl
    for r in range(1, _N // R):
        dr = d[r * R : (r + 1) * R, :]
        lt = dr < cval
        cval = jnp.minimum(dr, cval)
        cidx = jnp.where(lt, subl + r * R, cidx)
    cmin2 = jnp.min(cval, axis=0, keepdims=True)
    cidx_f = jnp.min(jnp.where(cval == cmin2, cidx, _N), axis=0)
    cmin_ref[0, 0, :] = cmin2[0]
    cidx_ref[0, 0, :] = cidx_f

    lane = jax.lax.broadcasted_iota(jnp.int32, (1, 128), 1)
    contrib = (
        jnp.sum(rmin) / _N
        + jnp.max(rmin) * _BETA
        + (_GAMMA + _DELTA * _N) * jnp.sum(cmin2) / _M
    ) * (1.0 / _B)
    contrib_vec = jnp.where(lane == 0, contrib, 0.0)

    @pl.when(b == 0)
    def _init():
        loss_ref[0, :, :] = contrib_vec

    @pl.when(b > 0)
    def _accum():
        loss_ref[0, :, :] = loss_ref[0, :, :] + contrib_vec


def kernel(xyz1, xyz2):
    xyz2t = xyz2.transpose(0, 2, 1)
    idx12_col, _cmin, cidx, loss_vec = pl.pallas_call(
        _cd_kernel,
        grid=(_B,),
        in_specs=[
            pl.BlockSpec((1, _N, 3), lambda b: (b, 0, 0)),
            pl.BlockSpec((1, 3, _M), lambda b: (b, 0, 0)),
        ],
        out_specs=[
            pl.BlockSpec((1, _N, 1), lambda b: (b, 0, 0)),
            pl.BlockSpec((1, 1, _M), lambda b: (b, 0, 0)),
            pl.BlockSpec((1, 1, _M), lambda b: (b, 0, 0)),
            pl.BlockSpec((1, 1, 128), lambda b: (0, 0, 0)),
        ],
        out_shape=[
            jax.ShapeDtypeStruct((_B, _N, 1), jnp.int32),
            jax.ShapeDtypeStruct((_B, 1, _M), jnp.float32),
            jax.ShapeDtypeStruct((_B, 1, _M), jnp.int32),
            jax.ShapeDtypeStruct((1, 1, 128), jnp.float32),
        ],
        compiler_params=pltpu.CompilerParams(
            dimension_semantics=("arbitrary",)
        ),
    )(xyz1, xyz2t)
    idx12 = idx12_col.reshape(_B, _N).astype(jnp.int64)
    idx21 = cidx.reshape(_B, _M).astype(jnp.int64)
    loss = loss_vec[0, 0, 0]
    return (loss, idx12, idx21)


# speedup vs baseline: 1.0125x; 1.0034x over previous
import jax
import jax.numpy as jnp
from jax.experimental import pallas as pl
from jax.experimental.pallas import tpu as pltpu

_B, _N, _M = 8, 2048, 4096
_BETA, _GAMMA, _DELTA = 1.0, 1.0, 0.0


def _cd_kernel(x1_ref, x2t_ref, idx12_ref, cidx_ref, loss_ref):
    b = pl.program_id(0)
    x1 = x1_ref[0]
    x2t = x2t_ref[0]
    a0, a1, a2 = x1[:, 0:1], x1[:, 1:2], x1[:, 2:3]
    aa = a0 * a0 + a1 * a1 + a2 * a2
    b0, b1, b2 = x2t[0:1, :], x2t[1:2, :], x2t[2:3, :]
    bb = b0 * b0 + b1 * b1 + b2 * b2
    ab = jax.lax.dot_general(
        x1, x2t, (((1,), (0,)), ((), ())), preferred_element_type=jnp.float32
    )
    d = aa + bb - 2.0 * ab

    C = 128
    lane128 = jax.lax.broadcasted_iota(jnp.int32, (_N, C), 1)
    rval = d[:, 0:C]
    ridx = lane128
    for c in range(1, _M // C):
        dc = d[:, c * C : (c + 1) * C]
        lt = dc < rval
        rval = jnp.minimum(dc, rval)
        ridx = jnp.where(lt, lane128 + c * C, ridx)
    rmin = jnp.min(rval, axis=1, keepdims=True)
    ridx_f = jnp.min(jnp.where(rval == rmin, ridx, _M), axis=1, keepdims=True)
    idx12_ref[0, :, :] = ridx_f

    R = 8
    subl = jax.lax.broadcasted_iota(jnp.int32, (R, _M), 0)
    cval = d[0:R, :]
    cidx = subl
    for r in range(1, _N // R):
        dr = d[r * R : (r + 1) * R, :]
        lt = dr < cval
        cval = jnp.minimum(dr, cval)
        cidx = jnp.where(lt, subl + r * R, cidx)
    cmin2 = jnp.min(cval, axis=0, keepdims=True)
    cidx_f = jnp.min(jnp.where(cval == cmin2, cidx, _N), axis=0)
    cidx_ref[0, 0, :] = cidx_f

    lane = jax.lax.broadcasted_iota(jnp.int32, (1, 128), 1)
    contrib = (
        jnp.sum(rmin) / _N
        + jnp.max(rmin) * _BETA
        + (_GAMMA + _DELTA * _N) * jnp.sum(cmin2) / _M
    ) * (1.0 / _B)
    contrib_vec = jnp.where(lane == 0, contrib, 0.0)

    @pl.when(b == 0)
    def _init():
        loss_ref[0, :, :] = contrib_vec

    @pl.when(b > 0)
    def _accum():
        loss_ref[0, :, :] = loss_ref[0, :, :] + contrib_vec


def kernel(xyz1, xyz2):
    xyz2t = xyz2.transpose(0, 2, 1)
    idx12_col, cidx, loss_vec = pl.pallas_call(
        _cd_kernel,
        grid=(_B,),
        in_specs=[
            pl.BlockSpec((1, _N, 3), lambda b: (b, 0, 0)),
            pl.BlockSpec((1, 3, _M), lambda b: (b, 0, 0)),
        ],
        out_specs=[
            pl.BlockSpec((1, _N, 1), lambda b: (b, 0, 0)),
            pl.BlockSpec((1, 1, _M), lambda b: (b, 0, 0)),
            pl.BlockSpec((1, 1, 128), lambda b: (0, 0, 0)),
        ],
        out_shape=[
            jax.ShapeDtypeStruct((_B, _N, 1), jnp.int32),
            jax.ShapeDtypeStruct((_B, 1, _M), jnp.int32),
            jax.ShapeDtypeStruct((1, 1, 128), jnp.float32),
        ],
        compiler_params=pltpu.CompilerParams(
            dimension_semantics=("arbitrary",)
        ),
    )(xyz1, xyz2t)
    idx12 = idx12_col.reshape(_B, _N).astype(jnp.int64)
    idx21 = cidx.reshape(_B, _M).astype(jnp.int64)
    loss = loss_vec[0, 0, 0]
    return (loss, idx12, idx21)


# speedup vs baseline: 1.0609x; 1.0479x over previous
import jax
import jax.numpy as jnp
from jax.experimental import pallas as pl
from jax.experimental.pallas import tpu as pltpu

_B, _N, _M = 8, 2048, 4096
_BETA, _GAMMA, _DELTA = 1.0, 1.0, 0.0


def _cd_kernel(x1_ref, x2t_ref, idx12_ref, cidx_ref, loss_ref):
    b = pl.program_id(0)
    x1 = x1_ref[0]
    x2t = x2t_ref[0]
    a0, a1, a2 = x1[:, 0:1], x1[:, 1:2], x1[:, 2:3]
    aa = a0 * a0 + a1 * a1 + a2 * a2
    b0, b1, b2 = x2t[0:1, :], x2t[1:2, :], x2t[2:3, :]
    bb = b0 * b0 + b1 * b1 + b2 * b2
    m2ab = jax.lax.dot_general(
        x1, x2t * (-2.0), (((1,), (0,)), ((), ())),
        preferred_element_type=jnp.float32,
    )
    d = aa + bb + m2ab

    C = 128
    lane128 = jax.lax.broadcasted_iota(jnp.int32, (_N, C), 1)
    rval = d[:, 0:C]
    ridx = lane128
    for c in range(1, _M // C):
        dc = d[:, c * C : (c + 1) * C]
        lt = dc < rval
        rval = jnp.minimum(dc, rval)
        ridx = jnp.where(lt, lane128 + c * C, ridx)
    rmin = jnp.min(rval, axis=1, keepdims=True)
    ridx_f = jnp.min(jnp.where(rval == rmin, ridx, _M), axis=1, keepdims=True)
    idx12_ref[0, :, :] = ridx_f

    R = 8
    subl = jax.lax.broadcasted_iota(jnp.int32, (R, _M), 0)
    cval = d[0:R, :]
    cidx = subl
    for r in range(1, _N // R):
        dr = d[r * R : (r + 1) * R, :]
        lt = dr < cval
        cval = jnp.minimum(dr, cval)
        cidx = jnp.where(lt, subl + r * R, cidx)
    cmin2 = jnp.min(cval, axis=0, keepdims=True)
    cidx_f = jnp.min(jnp.where(cval == cmin2, cidx, _N), axis=0)
    cidx_ref[0, 0, :] = cidx_f

    lane = jax.lax.broadcasted_iota(jnp.int32, (1, 128), 1)
    contrib = (
        jnp.sum(rmin) / _N
        + jnp.max(rmin) * _BETA
        + (_GAMMA + _DELTA * _N) * jnp.sum(cmin2) / _M
    ) * (1.0 / _B)
    contrib_vec = jnp.where(lane == 0, contrib, 0.0)

    @pl.when(b == 0)
    def _init():
        loss_ref[0, :, :] = contrib_vec

    @pl.when(b > 0)
    def _accum():
        loss_ref[0, :, :] = loss_ref[0, :, :] + contrib_vec


def kernel(xyz1, xyz2):
    xyz2t = xyz2.transpose(0, 2, 1)
    idx12_col, cidx, loss_vec = pl.pallas_call(
        _cd_kernel,
        grid=(_B,),
        in_specs=[
            pl.BlockSpec((1, _N, 3), lambda b: (b, 0, 0)),
            pl.BlockSpec((1, 3, _M), lambda b: (b, 0, 0)),
        ],
        out_specs=[
            pl.BlockSpec((1, _N, 1), lambda b: (b, 0, 0)),
            pl.BlockSpec((1, 1, _M), lambda b: (b, 0, 0)),
            pl.BlockSpec((1, 1, 128), lambda b: (0, 0, 0)),
        ],
        out_shape=[
            jax.ShapeDtypeStruct((_B, _N, 1), jnp.int32),
            jax.ShapeDtypeStruct((_B, 1, _M), jnp.int32),
            jax.ShapeDtypeStruct((1, 1, 128), jnp.float32),
        ],
        compiler_params=pltpu.CompilerParams(
            dimension_semantics=("arbitrary",)
        ),
    )(xyz1, xyz2t)
    idx12 = idx12_col.reshape(_B, _N).astype(jnp.int64)
    idx21 = cidx.reshape(_B, _M).astype(jnp.int64)
    loss = loss_vec[0, 0, 0]
    return (loss, idx12, idx21)
